# Initial kernel scaffold; baseline (speedup 1.0000x reference)
import jax
import jax.numpy as jnp
from jax.experimental import pallas as pl


def kernel(hcal, ecal, trck, edge_index, W1, b1, fc_W, fc_b):
    raise NotImplementedError("write your pallas kernel here")


# speedup vs baseline: 19.0547x; 19.0547x over previous
import functools

import jax
import jax.numpy as jnp
from jax import lax
from jax.experimental import pallas as pl
from jax.experimental.pallas import tpu as pltpu
from jax.experimental.pallas import tpu_sc as plsc

N = 100000
E = 1600000
B = 10
P = 10000
F = 64

NC = 2
NS = 16
NW = NC * NS
LANES = 16

N2 = 102400
SLICE = N2 // NS

E_PER_W = E // NW
CHUNK = 10000
NCH = E_PER_W // CHUNK

_MESH = dict(core_axis_name="c", subcore_axis_name="s",
             num_cores=NC, num_subcores=NS)


def _fill(ref, n, value):
    v = jnp.full((LANES,), value, jnp.float32)

    def body(i, _):
        ref[pl.ds(i * LANES, LANES)] = v
        return 0

    lax.fori_loop(0, n // LANES, body, 0)


def _sc_count(src, dst):
    mesh = plsc.VectorSubcoreMesh(**_MESH)

    @functools.partial(
        pl.kernel,
        out_type=jax.ShapeDtypeStruct((NC * 2 * N2,), jnp.float32),
        mesh=mesh,
        scratch_types=[
            pltpu.VMEM((CHUNK,), jnp.int32),
            pltpu.VMEM((CHUNK,), jnp.float32),
            pltpu.VMEM((SLICE,), jnp.float32),
            pltpu.VMEM_SHARED((N2,), jnp.float32),
            pltpu.VMEM_SHARED((N2,), jnp.float32),
        ],
    )
    def k(src_hbm, dst_hbm, out_hbm, idx_v, ones_v, zeros_v, acc_o, acc_i):
        cid = lax.axis_index("c")
        sid = lax.axis_index("s")
        _fill(ones_v, CHUNK, 1.0)
        _fill(zeros_v, SLICE, 0.0)
        sl = pl.ds(sid * SLICE, SLICE)
        pltpu.sync_copy(zeros_v, acc_o.at[sl])
        pltpu.sync_copy(zeros_v, acc_i.at[sl])
        plsc.subcore_barrier()

        ebase = (cid * NS + sid) * E_PER_W

        def chunk(i, _):
            base = pl.multiple_of(ebase + i * CHUNK, 8)
            pltpu.sync_copy(src_hbm.at[pl.ds(base, CHUNK)], idx_v)
            pltpu.sync_copy(ones_v, acc_o.at[idx_v], add=True)
            pltpu.sync_copy(dst_hbm.at[pl.ds(base, CHUNK)], idx_v)
            pltpu.sync_copy(ones_v, acc_i.at[idx_v], add=True)
            return 0

        lax.fori_loop(0, NCH, chunk, 0)
        plsc.subcore_barrier()
        obase = pl.multiple_of(cid * 2 * N2 + sid * SLICE, 8)
        pltpu.sync_copy(acc_o.at[sl], out_hbm.at[pl.ds(obase, SLICE)])
        pltpu.sync_copy(acc_i.at[sl], out_hbm.at[pl.ds(obase + N2, SLICE)])

    return k(src, dst).reshape(NC, 2, N2)


def _tc_feat(x3, cnt):
    def body(x_ref, c_ref, f_ref):
        deg_o = c_ref[0:1, :] + c_ref[2:3, :]
        so = lax.rsqrt(jnp.maximum(deg_o, 1.0))
        f_ref[...] = x_ref[...] * so

    return pl.pallas_call(
        body,
        out_shape=jax.ShapeDtypeStruct((3, N2), jnp.float32),
    )(x3, cnt)


def _sc_aggregate(src, dst, f0, f1, f2):
    mesh = plsc.VectorSubcoreMesh(**_MESH)

    @functools.partial(
        pl.kernel,
        out_type=jax.ShapeDtypeStruct((NC * 3 * N2,), jnp.float32),
        mesh=mesh,
        scratch_types=[
            pltpu.VMEM((CHUNK,), jnp.int32),
            pltpu.VMEM((CHUNK,), jnp.int32),
            pltpu.VMEM((CHUNK,), jnp.float32),
            pltpu.VMEM((CHUNK,), jnp.float32),
            pltpu.VMEM((CHUNK,), jnp.float32),
            pltpu.VMEM((SLICE,), jnp.float32),
            pltpu.VMEM_SHARED((N2,), jnp.float32),
            pltpu.VMEM_SHARED((N2,), jnp.float32),
            pltpu.VMEM_SHARED((N2,), jnp.float32),
            pltpu.SemaphoreType.DMA,
        ],
    )
    def k(src_hbm, dst_hbm, f0_hbm, f1_hbm, f2_hbm, out_hbm,
          src_v, dst_v, v0, v1, v2, zeros_v, a0, a1, a2, sem):
        cid = lax.axis_index("c")
        sid = lax.axis_index("s")
        _fill(zeros_v, SLICE, 0.0)
        sl = pl.ds(sid * SLICE, SLICE)
        pltpu.sync_copy(zeros_v, a0.at[sl])
        pltpu.sync_copy(zeros_v, a1.at[sl])
        pltpu.sync_copy(zeros_v, a2.at[sl])
        plsc.subcore_barrier()

        ebase = (cid * NS + sid) * E_PER_W

        def chunk(i, _):
            base = pl.multiple_of(ebase + i * CHUNK, 8)
            pltpu.sync_copy(src_hbm.at[pl.ds(base, CHUNK)], src_v)
            pltpu.sync_copy(dst_hbm.at[pl.ds(base, CHUNK)], dst_v)
            c0 = pltpu.async_copy(f0_hbm.at[src_v], v0, sem)
            c1 = pltpu.async_copy(f1_hbm.at[src_v], v1, sem)
            c2 = pltpu.async_copy(f2_hbm.at[src_v], v2, sem)
            c0.wait()
            c1.wait()
            c2.wait()
            pltpu.sync_copy(v0, a0.at[dst_v], add=True)
            pltpu.sync_copy(v1, a1.at[dst_v], add=True)
            pltpu.sync_copy(v2, a2.at[dst_v], add=True)
            return 0

        lax.fori_loop(0, NCH, chunk, 0)
        plsc.subcore_barrier()
        obase = pl.multiple_of(cid * 3 * N2 + sid * SLICE, 8)
        pltpu.sync_copy(a0.at[sl], out_hbm.at[pl.ds(obase, SLICE)])
        pltpu.sync_copy(a1.at[sl], out_hbm.at[pl.ds(obase + N2, SLICE)])
        pltpu.sync_copy(a2.at[sl], out_hbm.at[pl.ds(obase + 2 * N2, SLICE)])

    return k(src, dst, f0, f1, f2).reshape(NC, 3, N2)


P2 = 10240
Q = 2048
J = P2 // Q


def _tc_tail(agg4, deg4, w1t, b1, fcwt, fc_b):
    def body(a_ref, d_ref, w_ref, b_ref, fw_ref, fb_ref, o_ref):
        j = pl.program_id(1)
        s = a_ref[0, 0] + a_ref[0, 1]
        si = lax.rsqrt(jnp.maximum(d_ref[0, 0] + d_ref[0, 1], 1.0))
        a = s * si
        h = jnp.dot(w_ref[...], a, preferred_element_type=jnp.float32)
        h = jnp.maximum(h + b_ref[...], 0.0)
        part = jnp.sum(h * fw_ref[...])

        @pl.when(j == 0)
        def _():
            o_ref[0] = jnp.zeros((8, 128), jnp.float32)

        o_ref[0] = o_ref[0] + part

        @pl.when(j == J - 1)
        def _():
            o_ref[0] = jax.nn.sigmoid(o_ref[0] + fb_ref[0, 0])

    return pl.pallas_call(
        body,
        grid=(B, J),
        in_specs=[
            pl.BlockSpec((1, NC, 3, Q), lambda b, j: (b, 0, 0, j)),
            pl.BlockSpec((1, NC, 1, Q), lambda b, j: (b, 0, 0, j)),
            pl.BlockSpec((F, 3), lambda b, j: (0, 0)),
            pl.BlockSpec((F, 1), lambda b, j: (0, 0)),
            pl.BlockSpec((F, Q), lambda b, j: (0, j)),
            pl.BlockSpec((1, 1), lambda b, j: (0, 0)),
        ],
        out_specs=pl.BlockSpec((1, 8, 128), lambda b, j: (b, 0, 0)),
        out_shape=jax.ShapeDtypeStruct((B, 8, 128), jnp.float32),
    )(agg4, deg4, w1t, b1, fcwt, fc_b)


def kernel(hcal, ecal, trck, edge_index, W1, b1, fc_W, fc_b):
    ei = edge_index.astype(jnp.int32)
    src = ei[0]
    dst = ei[1]

    x3 = jnp.zeros((3, N2), jnp.float32)
    x3 = x3.at[0, :N].set(hcal).at[1, :N].set(ecal).at[2, :N].set(trck)

    cnt = _sc_count(src, dst)
    feat = _tc_feat(x3, cnt.reshape(NC * 2, N2))
    aggp = _sc_aggregate(src, dst, feat[0], feat[1], feat[2])

    agg4 = aggp[:, :, :N].reshape(NC, 3, B, P).transpose(2, 0, 1, 3)
    agg4 = jnp.pad(agg4, ((0, 0), (0, 0), (0, 0), (0, P2 - P)))
    deg4 = cnt[:, 1, :N].reshape(NC, 1, B, P).transpose(2, 0, 1, 3)
    deg4 = jnp.pad(deg4, ((0, 0), (0, 0), (0, 0), (0, P2 - P)))
    w1t = W1.T
    b1c = b1.reshape(F, 1)
    fcwt = fc_W.reshape(P, F).T
    fcwt = jnp.pad(fcwt, ((0, 0), (0, P2 - P)))
    fcb = fc_b.reshape(1, 1)
    out = _tc_tail(agg4, deg4, w1t, b1c, fcwt, fcb)
    return out[:, 0, 0:1]


# speedup vs baseline: 21.3465x; 1.1203x over previous
import functools

import jax
import jax.numpy as jnp
from jax import lax
from jax.experimental import pallas as pl
from jax.experimental.pallas import tpu as pltpu
from jax.experimental.pallas import tpu_sc as plsc

N = 100000
E = 1600000
B = 10
P = 10000
F = 64

NC = 2
NS = 16
NW = NC * NS
LANES = 16

N2 = 102400
SLICE = N2 // NS

E_PER_W = E // NW
CHUNK = 5000
NCH = E_PER_W // CHUNK

_MESH = dict(core_axis_name="c", subcore_axis_name="s",
             num_cores=NC, num_subcores=NS)


def _fill(ref, n, value):
    v = jnp.full((LANES,), value, jnp.float32)

    def body(i, _):
        ref[pl.ds(i * LANES, LANES)] = v
        return 0

    lax.fori_loop(0, n // LANES, body, 0)


def _sc_count(src, dst):
    mesh = plsc.VectorSubcoreMesh(**_MESH)

    @functools.partial(
        pl.kernel,
        out_type=jax.ShapeDtypeStruct((NC * 2 * N2,), jnp.float32),
        mesh=mesh,
        scratch_types=[
            [pltpu.VMEM((CHUNK,), jnp.int32) for _ in range(3)],
            [pltpu.VMEM((CHUNK,), jnp.int32) for _ in range(3)],
            pltpu.VMEM((CHUNK,), jnp.float32),
            pltpu.VMEM((SLICE,), jnp.float32),
            pltpu.VMEM_SHARED((N2,), jnp.float32),
            pltpu.VMEM_SHARED((N2,), jnp.float32),
            [pltpu.SemaphoreType.DMA for _ in range(3)],
            [pltpu.SemaphoreType.DMA for _ in range(3)],
        ],
    )
    def k(src_hbm, dst_hbm, out_hbm, srcb, dstb, ones_v, zeros_v,
          acc_o, acc_i, lsem, ssem):
        cid = lax.axis_index("c")
        sid = lax.axis_index("s")
        _fill(ones_v, CHUNK, 1.0)
        _fill(zeros_v, SLICE, 0.0)
        sl = pl.ds(sid * SLICE, SLICE)
        pltpu.sync_copy(zeros_v, acc_o.at[sl])
        pltpu.sync_copy(zeros_v, acc_i.at[sl])
        plsc.subcore_barrier()

        ebase = (cid * NS + sid) * E_PER_W

        def load(i):
            b = i % 3
            base = pl.multiple_of(ebase + i * CHUNK, 8)
            return [
                pltpu.async_copy(src_hbm.at[pl.ds(base, CHUNK)], srcb[b], lsem[b]),
                pltpu.async_copy(dst_hbm.at[pl.ds(base, CHUNK)], dstb[b], lsem[b]),
            ]

        def scat(i):
            b = i % 3
            return [
                pltpu.async_copy(ones_v, acc_o.at[srcb[b]], ssem[b], add=True),
                pltpu.async_copy(ones_v, acc_i.at[dstb[b]], ssem[b], add=True),
            ]

        ld = [None] * NCH
        st = [None] * NCH
        ld[0] = load(0)
        ld[1] = load(1)
        for i in range(NCH):
            for c in ld[i]:
                c.wait()
            st[i] = scat(i)
            if i >= 1:
                for c in st[i - 1]:
                    c.wait()
            if i + 2 < NCH:
                ld[i + 2] = load(i + 2)
        for c in st[NCH - 1]:
            c.wait()

        plsc.subcore_barrier()
        obase = pl.multiple_of(cid * 2 * N2 + sid * SLICE, 8)
        pltpu.sync_copy(acc_o.at[sl], out_hbm.at[pl.ds(obase, SLICE)])
        pltpu.sync_copy(acc_i.at[sl], out_hbm.at[pl.ds(obase + N2, SLICE)])

    return k(src, dst).reshape(NC, 2, N2)


def _tc_feat(x3, cnt):
    def body(x_ref, c_ref, f_ref):
        deg_o = c_ref[0:1, :] + c_ref[2:3, :]
        so = lax.rsqrt(jnp.maximum(deg_o, 1.0))
        f_ref[...] = x_ref[...] * so

    return pl.pallas_call(
        body,
        out_shape=jax.ShapeDtypeStruct((3, N2), jnp.float32),
    )(x3, cnt)


def _sc_aggregate(src, dst, f0, f1, f2):
    mesh = plsc.VectorSubcoreMesh(**_MESH)

    @functools.partial(
        pl.kernel,
        out_type=jax.ShapeDtypeStruct((NC * 3 * N2,), jnp.float32),
        mesh=mesh,
        scratch_types=[
            [pltpu.VMEM((CHUNK,), jnp.int32) for _ in range(2)],
            [pltpu.VMEM((CHUNK,), jnp.int32) for _ in range(2)],
            [[pltpu.VMEM((CHUNK,), jnp.float32) for _ in range(2)]
             for _ in range(3)],
            pltpu.VMEM((SLICE,), jnp.float32),
            [pltpu.VMEM_SHARED((N2,), jnp.float32) for _ in range(3)],
            [pltpu.SemaphoreType.DMA for _ in range(2)],
            [pltpu.SemaphoreType.DMA for _ in range(2)],
            [pltpu.SemaphoreType.DMA for _ in range(2)],
        ],
    )
    def k(src_hbm, dst_hbm, f0_hbm, f1_hbm, f2_hbm, out_hbm,
          srcb, dstb, vb, zeros_v, acc, lsem, gsem, ssem):
        cid = lax.axis_index("c")
        sid = lax.axis_index("s")
        fh = [f0_hbm, f1_hbm, f2_hbm]
        _fill(zeros_v, SLICE, 0.0)
        sl = pl.ds(sid * SLICE, SLICE)
        for a in acc:
            pltpu.sync_copy(zeros_v, a.at[sl])
        plsc.subcore_barrier()

        ebase = (cid * NS + sid) * E_PER_W

        def load(i):
            b = i % 2
            base = pl.multiple_of(ebase + i * CHUNK, 8)
            return [
                pltpu.async_copy(src_hbm.at[pl.ds(base, CHUNK)], srcb[b], lsem[b]),
                pltpu.async_copy(dst_hbm.at[pl.ds(base, CHUNK)], dstb[b], lsem[b]),
            ]

        def gather(i):
            b = i % 2
            return [pltpu.async_copy(fh[kk].at[srcb[b]], vb[kk][b], gsem[b])
                    for kk in range(3)]

        def scat(i):
            b = i % 2
            return [pltpu.async_copy(vb[kk][b], acc[kk].at[dstb[b]], ssem[b],
                                     add=True)
                    for kk in range(3)]

        ld = [None] * NCH
        gt = [None] * NCH
        st = [None] * NCH
        ld[0] = load(0)
        for c in ld[0]:
            c.wait()
        gt[0] = gather(0)
        if NCH > 1:
            ld[1] = load(1)
        for i in range(NCH):
            for c in gt[i]:
                c.wait()
            st[i] = scat(i)
            if i + 1 < NCH:
                for c in ld[i + 1]:
                    c.wait()
                gt[i + 1] = gather(i + 1)
            for c in st[i]:
                c.wait()
            if i + 2 < NCH:
                ld[i + 2] = load(i + 2)

        plsc.subcore_barrier()
        obase = pl.multiple_of(cid * 3 * N2 + sid * SLICE, 8)
        for kk in range(3):
            pltpu.sync_copy(acc[kk].at[sl],
                            out_hbm.at[pl.ds(obase + kk * N2, SLICE)])

    return k(src, dst, f0, f1, f2).reshape(NC, 3, N2)


P2 = 10240
Q = 2048
J = P2 // Q


def _tc_tail(agg4, deg4, w1t, b1, fcwt, fc_b):
    def body(a_ref, d_ref, w_ref, b_ref, fw_ref, fb_ref, o_ref):
        j = pl.program_id(1)
        s = a_ref[0] + a_ref[1]
        si = lax.rsqrt(jnp.maximum(d_ref[0] + d_ref[1], 1.0))
        a = s * si
        h = jnp.dot(w_ref[...], a, preferred_element_type=jnp.float32)
        h = jnp.maximum(h + b_ref[...], 0.0)
        part = jnp.sum(h * fw_ref[...])

        @pl.when(j == 0)
        def _():
            o_ref[0] = jnp.zeros((8, 128), jnp.float32)

        o_ref[0] = o_ref[0] + part

        @pl.when(j == J - 1)
        def _():
            o_ref[0] = jax.nn.sigmoid(o_ref[0] + fb_ref[0, 0])

    return pl.pallas_call(
        body,
        grid=(B, J),
        in_specs=[
            pl.BlockSpec((NC, 3, Q), lambda b, j: (0, 0, b * J + j)),
            pl.BlockSpec((NC, 1, Q), lambda b, j: (0, 0, b * J + j)),
            pl.BlockSpec((F, 3), lambda b, j: (0, 0)),
            pl.BlockSpec((F, 1), lambda b, j: (0, 0)),
            pl.BlockSpec((F, Q), lambda b, j: (0, j)),
            pl.BlockSpec((1, 1), lambda b, j: (0, 0)),
        ],
        out_specs=pl.BlockSpec((1, 8, 128), lambda b, j: (b, 0, 0)),
        out_shape=jax.ShapeDtypeStruct((B, 8, 128), jnp.float32),
    )(agg4, deg4, w1t, b1, fcwt, fc_b)


def kernel(hcal, ecal, trck, edge_index, W1, b1, fc_W, fc_b):
    ei = edge_index.astype(jnp.int32)
    src = ei[0]
    dst = ei[1]

    x3 = jnp.zeros((3, N2), jnp.float32)
    x3 = x3.at[0, :N].set(hcal).at[1, :N].set(ecal).at[2, :N].set(trck)

    cnt = _sc_count(src, dst)
    feat = _tc_feat(x3, cnt.reshape(NC * 2, N2))
    aggp = _sc_aggregate(src, dst, feat[0], feat[1], feat[2])

    agg4 = jnp.pad(aggp[:, :, :N].reshape(NC, 3, B, P),
                   ((0, 0), (0, 0), (0, 0), (0, P2 - P)))
    agg4 = agg4.reshape(NC, 3, B * P2)
    deg4 = jnp.pad(cnt[:, 1:2, :N].reshape(NC, 1, B, P),
                   ((0, 0), (0, 0), (0, 0), (0, P2 - P)))
    deg4 = deg4.reshape(NC, 1, B * P2)
    w1t = W1.T
    b1c = b1.reshape(F, 1)
    fcwt = fc_W.reshape(P, F).T
    fcwt = jnp.pad(fcwt, ((0, 0), (0, P2 - P)))
    fcb = fc_b.reshape(1, 1)
    out = _tc_tail(agg4, deg4, w1t, b1c, fcwt, fcb)
    return out[:, 0, 0:1]


# speedup vs baseline: 21.6016x; 1.0120x over previous
import functools

import jax
import jax.numpy as jnp
from jax import lax
from jax.experimental import pallas as pl
from jax.experimental.pallas import tpu as pltpu
from jax.experimental.pallas import tpu_sc as plsc

N = 100000
E = 1600000
B = 10
P = 10000
F = 64

NC = 2
NS = 16
NW = NC * NS
LANES = 16

N2 = 102400
SLICE = N2 // NS

E_PER_W = E // NW
CHUNK = 5000
NCH = E_PER_W // CHUNK

_MESH = dict(core_axis_name="c", subcore_axis_name="s",
             num_cores=NC, num_subcores=NS)


def _fill(ref, n, value):
    v = jnp.full((LANES,), value, jnp.float32)

    def body(i, _):
        ref[pl.ds(i * LANES, LANES)] = v
        return 0

    lax.fori_loop(0, n // LANES, body, 0)


def _sc_count(src, dst):
    mesh = plsc.VectorSubcoreMesh(**_MESH)

    @functools.partial(
        pl.kernel,
        out_type=jax.ShapeDtypeStruct((NC * 2 * N2,), jnp.float32),
        mesh=mesh,
        scratch_types=[
            [pltpu.VMEM((CHUNK,), jnp.int32) for _ in range(3)],
            [pltpu.VMEM((CHUNK,), jnp.int32) for _ in range(3)],
            pltpu.VMEM((CHUNK,), jnp.float32),
            pltpu.VMEM((SLICE,), jnp.float32),
            pltpu.VMEM_SHARED((N2,), jnp.float32),
            pltpu.VMEM_SHARED((N2,), jnp.float32),
            [pltpu.SemaphoreType.DMA for _ in range(3)],
            [pltpu.SemaphoreType.DMA for _ in range(3)],
        ],
    )
    def k(src_hbm, dst_hbm, out_hbm, srcb, dstb, ones_v, zeros_v,
          acc_o, acc_i, lsem, ssem):
        cid = lax.axis_index("c")
        sid = lax.axis_index("s")
        _fill(ones_v, CHUNK, 1.0)
        _fill(zeros_v, SLICE, 0.0)
        sl = pl.ds(sid * SLICE, SLICE)
        pltpu.sync_copy(zeros_v, acc_o.at[sl])
        pltpu.sync_copy(zeros_v, acc_i.at[sl])
        plsc.subcore_barrier()

        ebase = (cid * NS + sid) * E_PER_W

        def load(i):
            b = i % 3
            base = pl.multiple_of(ebase + i * CHUNK, 8)
            return [
                pltpu.async_copy(src_hbm.at[pl.ds(base, CHUNK)], srcb[b], lsem[b]),
                pltpu.async_copy(dst_hbm.at[pl.ds(base, CHUNK)], dstb[b], lsem[b]),
            ]

        def scat(i):
            b = i % 3
            return [
                pltpu.async_copy(ones_v, acc_o.at[srcb[b]], ssem[b], add=True),
                pltpu.async_copy(ones_v, acc_i.at[dstb[b]], ssem[b], add=True),
            ]

        ld = [None] * NCH
        st = [None] * NCH
        ld[0] = load(0)
        ld[1] = load(1)
        for i in range(NCH):
            for c in ld[i]:
                c.wait()
            st[i] = scat(i)
            if i >= 1:
                for c in st[i - 1]:
                    c.wait()
            if i + 2 < NCH:
                ld[i + 2] = load(i + 2)
        for c in st[NCH - 1]:
            c.wait()

        plsc.subcore_barrier()
        obase = pl.multiple_of(cid * 2 * N2 + sid * SLICE, 8)
        pltpu.sync_copy(acc_o.at[sl], out_hbm.at[pl.ds(obase, SLICE)])
        pltpu.sync_copy(acc_i.at[sl], out_hbm.at[pl.ds(obase + N2, SLICE)])

    return k(src, dst).reshape(NC, 2, N2)


def _tc_feat(x3, cnt):
    def body(x_ref, c_ref, f_ref):
        deg_o = c_ref[0:1, :] + c_ref[2:3, :]
        so = lax.rsqrt(jnp.maximum(deg_o, 1.0))
        f_ref[...] = x_ref[...] * so

    return pl.pallas_call(
        body,
        out_shape=jax.ShapeDtypeStruct((3, N2), jnp.float32),
    )(x3, cnt)


def _sc_aggregate(src, dst, f0, f1, f2):
    mesh = plsc.VectorSubcoreMesh(**_MESH)

    @functools.partial(
        pl.kernel,
        out_type=jax.ShapeDtypeStruct((NC * 3 * N2,), jnp.float32),
        mesh=mesh,
        scratch_types=[
            [pltpu.VMEM((CHUNK,), jnp.int32) for _ in range(2)],
            [pltpu.VMEM((CHUNK,), jnp.int32) for _ in range(2)],
            [[pltpu.VMEM((CHUNK,), jnp.float32) for _ in range(2)]
             for _ in range(3)],
            pltpu.VMEM((SLICE,), jnp.float32),
            [pltpu.VMEM_SHARED((N2,), jnp.float32) for _ in range(3)],
            [pltpu.SemaphoreType.DMA for _ in range(2)],
            [pltpu.SemaphoreType.DMA for _ in range(2)],
            [pltpu.SemaphoreType.DMA for _ in range(2)],
        ],
    )
    def k(src_hbm, dst_hbm, f0_hbm, f1_hbm, f2_hbm, out_hbm,
          srcb, dstb, vb, zeros_v, acc, lsem, gsem, ssem):
        cid = lax.axis_index("c")
        sid = lax.axis_index("s")
        fh = [f0_hbm, f1_hbm, f2_hbm]
        _fill(zeros_v, SLICE, 0.0)
        sl = pl.ds(sid * SLICE, SLICE)
        for a in acc:
            pltpu.sync_copy(zeros_v, a.at[sl])
        plsc.subcore_barrier()

        ebase = (cid * NS + sid) * E_PER_W

        def load(i):
            b = i % 2
            base = pl.multiple_of(ebase + i * CHUNK, 8)
            return [
                pltpu.async_copy(src_hbm.at[pl.ds(base, CHUNK)], srcb[b], lsem[b]),
                pltpu.async_copy(dst_hbm.at[pl.ds(base, CHUNK)], dstb[b], lsem[b]),
            ]

        def gather(i):
            b = i % 2
            return [pltpu.async_copy(fh[kk].at[srcb[b]], vb[kk][b], gsem[b])
                    for kk in range(3)]

        def scat(i):
            b = i % 2
            return [pltpu.async_copy(vb[kk][b], acc[kk].at[dstb[b]], ssem[b],
                                     add=True)
                    for kk in range(3)]

        ld = [None] * NCH
        gt = [None] * NCH
        st = [None] * NCH
        ld[0] = load(0)
        for c in ld[0]:
            c.wait()
        gt[0] = gather(0)
        if NCH > 1:
            ld[1] = load(1)
        for i in range(NCH):
            for c in gt[i]:
                c.wait()
            st[i] = scat(i)
            if i + 1 < NCH:
                for c in ld[i + 1]:
                    c.wait()
                gt[i + 1] = gather(i + 1)
            for c in st[i]:
                c.wait()
            if i + 2 < NCH:
                ld[i + 2] = load(i + 2)

        plsc.subcore_barrier()
        obase = pl.multiple_of(cid * 3 * N2 + sid * SLICE, 8)
        for kk in range(3):
            pltpu.sync_copy(acc[kk].at[sl],
                            out_hbm.at[pl.ds(obase + kk * N2, SLICE)])

    return k(src, dst, f0, f1, f2).reshape(NC, 3, N2)


P2 = 10240
Q = 2048
J = P2 // Q


def _tc_tail(agg4, deg4, w1t, b1, fcwt, fc_b):
    def body(a_ref, d_ref, w_ref, b_ref, fw_ref, fb_ref, o_ref):
        j = pl.program_id(0)
        b = pl.program_id(1)
        s = a_ref[0] + a_ref[1]
        si = lax.rsqrt(jnp.maximum(d_ref[0] + d_ref[1], 1.0))
        a = s * si
        h = jnp.dot(w_ref[...], a, preferred_element_type=jnp.float32)
        h = jnp.maximum(h + b_ref[...], 0.0)
        part = jnp.sum(h * fw_ref[...])

        @pl.when(j == 0)
        def _():
            o_ref[b] = jnp.zeros((8, 128), jnp.float32)

        o_ref[b] = o_ref[b] + part

        @pl.when(j == J - 1)
        def _():
            o_ref[b] = jax.nn.sigmoid(o_ref[b] + fb_ref[0, 0])

    return pl.pallas_call(
        body,
        grid=(J, B),
        in_specs=[
            pl.BlockSpec((NC, 3, Q), lambda j, b: (0, 0, b * J + j)),
            pl.BlockSpec((NC, 1, Q), lambda j, b: (0, 0, b * J + j)),
            pl.BlockSpec((F, 3), lambda j, b: (0, 0)),
            pl.BlockSpec((F, 1), lambda j, b: (0, 0)),
            pl.BlockSpec((F, Q), lambda j, b: (0, j)),
            pl.BlockSpec((1, 1), lambda j, b: (0, 0)),
        ],
        out_specs=pl.BlockSpec((B, 8, 128), lambda j, b: (0, 0, 0)),
        out_shape=jax.ShapeDtypeStruct((B, 8, 128), jnp.float32),
    )(agg4, deg4, w1t, b1, fcwt, fc_b)


def kernel(hcal, ecal, trck, edge_index, W1, b1, fc_W, fc_b):
    if edge_index.dtype == jnp.int32:
        src = edge_index[0]
        dst = edge_index[1]
    else:
        lo = jax.lax.bitcast_convert_type(edge_index, jnp.int32)
        src = lo[0, :, 0]
        dst = lo[1, :, 0]

    x3 = jnp.zeros((3, N2), jnp.float32)
    x3 = x3.at[0, :N].set(hcal).at[1, :N].set(ecal).at[2, :N].set(trck)

    cnt = _sc_count(src, dst)
    feat = _tc_feat(x3, cnt.reshape(NC * 2, N2))
    aggp = _sc_aggregate(src, dst, feat[0], feat[1], feat[2])

    agg4 = jnp.pad(aggp[:, :, :N].reshape(NC, 3, B, P),
                   ((0, 0), (0, 0), (0, 0), (0, P2 - P)))
    agg4 = agg4.reshape(NC, 3, B * P2)
    deg4 = jnp.pad(cnt[:, 1:2, :N].reshape(NC, 1, B, P),
                   ((0, 0), (0, 0), (0, 0), (0, P2 - P)))
    deg4 = deg4.reshape(NC, 1, B * P2)
    w1t = W1.T
    b1c = b1.reshape(F, 1)
    fcwt = fc_W.reshape(P, F).T
    fcwt = jnp.pad(fcwt, ((0, 0), (0, P2 - P)))
    fcb = fc_b.reshape(1, 1)
    out = _tc_tail(agg4, deg4, w1t, b1c, fcwt, fcb)
    return out[:, 0, 0:1]


# speedup vs baseline: 25.3053x; 1.1715x over previous
import functools

import jax
import jax.numpy as jnp
from jax import lax
from jax.experimental import pallas as pl
from jax.experimental.pallas import tpu as pltpu
from jax.experimental.pallas import tpu_sc as plsc

N = 100000
E = 1600000
B = 10
P = 10000
F = 64

NC = 2
NS = 16
NW = NC * NS
LANES = 16

N2 = 102400
SLICE = N2 // NS

E_PER_W = E // NW
CHUNK = 5000
NCH = E_PER_W // CHUNK

_MESH = dict(core_axis_name="c", subcore_axis_name="s",
             num_cores=NC, num_subcores=NS)


def _fill(ref, n, value):
    v = jnp.full((LANES,), value, jnp.float32)

    def body(i, _):
        ref[pl.ds(i * LANES, LANES)] = v
        return 0

    lax.fori_loop(0, n // LANES, body, 0)


def _sc_count(edges):
    mesh = plsc.VectorSubcoreMesh(**_MESH)

    @functools.partial(
        pl.kernel,
        out_type=jax.ShapeDtypeStruct((NC * 2 * N2,), jnp.float32),
        mesh=mesh,
        scratch_types=[
            [pltpu.VMEM((CHUNK,), jnp.int32) for _ in range(3)],
            [pltpu.VMEM((CHUNK,), jnp.int32) for _ in range(3)],
            pltpu.VMEM((CHUNK,), jnp.float32),
            pltpu.VMEM((SLICE,), jnp.float32),
            pltpu.VMEM_SHARED((N2,), jnp.float32),
            pltpu.VMEM_SHARED((N2,), jnp.float32),
            [pltpu.SemaphoreType.DMA for _ in range(3)],
            [pltpu.SemaphoreType.DMA for _ in range(3)],
        ],
    )
    def k(edges_hbm, out_hbm, srcb, dstb, ones_v, zeros_v,
          acc_o, acc_i, lsem, ssem):
        cid = lax.axis_index("c")
        sid = lax.axis_index("s")
        _fill(ones_v, CHUNK, 1.0)
        _fill(zeros_v, SLICE, 0.0)
        sl = pl.ds(sid * SLICE, SLICE)
        pltpu.sync_copy(zeros_v, acc_o.at[sl])
        pltpu.sync_copy(zeros_v, acc_i.at[sl])
        plsc.subcore_barrier()

        ebase = (cid * NS + sid) * E_PER_W

        def load(i):
            b = i % 3
            base = pl.multiple_of(ebase + i * CHUNK, 8)
            return [
                pltpu.async_copy(edges_hbm.at[pl.ds(base, CHUNK)], srcb[b],
                                 lsem[b]),
                pltpu.async_copy(edges_hbm.at[pl.ds(base + E, CHUNK)], dstb[b],
                                 lsem[b]),
            ]

        def scat(i):
            b = i % 3
            return [
                pltpu.async_copy(ones_v, acc_o.at[srcb[b]], ssem[b], add=True),
                pltpu.async_copy(ones_v, acc_i.at[dstb[b]], ssem[b], add=True),
            ]

        ld = [None] * NCH
        st = [None] * NCH
        ld[0] = load(0)
        ld[1] = load(1)
        for i in range(NCH):
            for c in ld[i]:
                c.wait()
            st[i] = scat(i)
            if i >= 1:
                for c in st[i - 1]:
                    c.wait()
            if i + 2 < NCH:
                ld[i + 2] = load(i + 2)
        for c in st[NCH - 1]:
            c.wait()

        plsc.subcore_barrier()
        obase = pl.multiple_of(cid * 2 * N2 + sid * SLICE, 8)
        pltpu.sync_copy(acc_o.at[sl], out_hbm.at[pl.ds(obase, SLICE)])
        pltpu.sync_copy(acc_i.at[sl], out_hbm.at[pl.ds(obase + N2, SLICE)])

    return k(edges).reshape(NC, 2, N2)


def _tc_feat(x3, cnt):
    def body(x_ref, c_ref, f_ref):
        deg_o = c_ref[0:1, :] + c_ref[2:3, :]
        so = lax.rsqrt(jnp.maximum(deg_o, 1.0))
        f_ref[...] = x_ref[...] * so

    return pl.pallas_call(
        body,
        out_shape=jax.ShapeDtypeStruct((3, N2), jnp.float32),
    )(x3, cnt)


def _sc_aggregate(edges, f0, f1, f2):
    mesh = plsc.VectorSubcoreMesh(**_MESH)

    @functools.partial(
        pl.kernel,
        out_type=jax.ShapeDtypeStruct((NC * 3 * N2,), jnp.float32),
        mesh=mesh,
        scratch_types=[
            [pltpu.VMEM((CHUNK,), jnp.int32) for _ in range(3)],
            [pltpu.VMEM((CHUNK,), jnp.int32) for _ in range(3)],
            [[pltpu.VMEM((CHUNK,), jnp.float32) for _ in range(3)]
             for _ in range(3)],
            pltpu.VMEM((SLICE,), jnp.float32),
            [pltpu.VMEM_SHARED((N2,), jnp.float32) for _ in range(3)],
            [pltpu.SemaphoreType.DMA for _ in range(3)],
            [pltpu.SemaphoreType.DMA for _ in range(3)],
            [pltpu.SemaphoreType.DMA for _ in range(3)],
        ],
    )
    def k(edges_hbm, f0_hbm, f1_hbm, f2_hbm, out_hbm,
          srcb, dstb, vb, zeros_v, acc, lsem, gsem, ssem):
        cid = lax.axis_index("c")
        sid = lax.axis_index("s")
        fh = [f0_hbm, f1_hbm, f2_hbm]
        _fill(zeros_v, SLICE, 0.0)
        sl = pl.ds(sid * SLICE, SLICE)
        for a in acc:
            pltpu.sync_copy(zeros_v, a.at[sl])
        plsc.subcore_barrier()

        ebase = (cid * NS + sid) * E_PER_W

        def load(i):
            b = i % 3
            base = pl.multiple_of(ebase + i * CHUNK, 8)
            return [
                pltpu.async_copy(edges_hbm.at[pl.ds(base, CHUNK)], srcb[b],
                                 lsem[b]),
                pltpu.async_copy(edges_hbm.at[pl.ds(base + E, CHUNK)], dstb[b],
                                 lsem[b]),
            ]

        def gather(i):
            b = i % 3
            return [pltpu.async_copy(fh[kk].at[srcb[b]], vb[kk][b], gsem[b])
                    for kk in range(3)]

        def scat(i):
            b = i % 3
            return [pltpu.async_copy(vb[kk][b], acc[kk].at[dstb[b]], ssem[b],
                                     add=True)
                    for kk in range(3)]

        ld = [None] * NCH
        gt = [None] * NCH
        st = [None] * NCH
        ld[0] = load(0)
        for c in ld[0]:
            c.wait()
        gt[0] = gather(0)
        if NCH > 1:
            ld[1] = load(1)
        for i in range(NCH):
            for c in gt[i]:
                c.wait()
            st[i] = scat(i)
            if i + 1 < NCH:
                for c in ld[i + 1]:
                    c.wait()
                gt[i + 1] = gather(i + 1)
            if i >= 1:
                for c in st[i - 1]:
                    c.wait()
            if i + 2 < NCH:
                ld[i + 2] = load(i + 2)
        for c in st[NCH - 1]:
            c.wait()

        plsc.subcore_barrier()
        obase = pl.multiple_of(cid * 3 * N2 + sid * SLICE, 8)
        for kk in range(3):
            pltpu.sync_copy(acc[kk].at[sl],
                            out_hbm.at[pl.ds(obase + kk * N2, SLICE)])

    return k(edges, f0, f1, f2).reshape(NC, 3, N2)


P2 = 10240
Q = 2048
J = P2 // Q


def _tc_tail(agg4, deg4, w1t, b1, fcwt, fc_b):
    def body(a_ref, d_ref, w_ref, b_ref, fw_ref, fb_ref, o_ref):
        b = pl.program_id(0)
        s = a_ref[0] + a_ref[1]
        si = lax.rsqrt(jnp.maximum(d_ref[0] + d_ref[1], 1.0))
        a = s * si
        h = jnp.dot(w_ref[...], a, preferred_element_type=jnp.float32)
        h = jnp.maximum(h + b_ref[...], 0.0)
        val = jnp.sum(h * fw_ref[...])
        o_ref[b] = jnp.full((8, 128), jax.nn.sigmoid(val + fb_ref[0, 0]),
                            jnp.float32)

    return pl.pallas_call(
        body,
        grid=(B,),
        in_specs=[
            pl.BlockSpec((NC, 3, P2), lambda b: (0, 0, b)),
            pl.BlockSpec((NC, 1, P2), lambda b: (0, 0, b)),
            pl.BlockSpec((F, 3), lambda b: (0, 0)),
            pl.BlockSpec((F, 1), lambda b: (0, 0)),
            pl.BlockSpec((F, P2), lambda b: (0, 0)),
            pl.BlockSpec((1, 1), lambda b: (0, 0)),
        ],
        out_specs=pl.BlockSpec((B, 8, 128), lambda b: (0, 0, 0)),
        out_shape=jax.ShapeDtypeStruct((B, 8, 128), jnp.float32),
    )(agg4, deg4, w1t, b1, fcwt, fc_b)


def kernel(hcal, ecal, trck, edge_index, W1, b1, fc_W, fc_b):
    if edge_index.dtype != jnp.int32:
        edge_index = edge_index.astype(jnp.int32)
    edges = edge_index.reshape(2 * E)

    x3 = jnp.zeros((3, N2), jnp.float32)
    x3 = x3.at[0, :N].set(hcal).at[1, :N].set(ecal).at[2, :N].set(trck)

    cnt = _sc_count(edges)
    feat = _tc_feat(x3, cnt.reshape(NC * 2, N2))
    aggp = _sc_aggregate(edges, feat[0], feat[1], feat[2])

    agg4 = jnp.pad(aggp[:, :, :N].reshape(NC, 3, B, P),
                   ((0, 0), (0, 0), (0, 0), (0, P2 - P)))
    agg4 = agg4.reshape(NC, 3, B * P2)
    deg4 = jnp.pad(cnt[:, 1:2, :N].reshape(NC, 1, B, P),
                   ((0, 0), (0, 0), (0, 0), (0, P2 - P)))
    deg4 = deg4.reshape(NC, 1, B * P2)
    w1t = W1.T
    b1c = b1.reshape(F, 1)
    fcwt = fc_W.reshape(P, F).T
    fcwt = jnp.pad(fcwt, ((0, 0), (0, P2 - P)))
    fcb = fc_b.reshape(1, 1)
    out = _tc_tail(agg4, deg4, w1t, b1c, fcwt, fcb)
    return out[:, 0, 0:1]


# speedup vs baseline: 37.7560x; 1.4920x over previous
import functools

import jax
import jax.numpy as jnp
from jax import lax
from jax.experimental import pallas as pl
from jax.experimental.pallas import tpu as pltpu
from jax.experimental.pallas import tpu_sc as plsc

N = 100000
E = 1600000
B = 10
P = 10000
F = 64

NC = 2
NS = 16
NW = NC * NS
LANES = 16

N2 = 102400
SLICE = N2 // NS

E_PER_W = E // NW
CHUNK = 5000
NCH = E_PER_W // CHUNK

_MESH = dict(core_axis_name="c", subcore_axis_name="s",
             num_cores=NC, num_subcores=NS)


def _fill(ref, n, value):
    v = jnp.full((LANES,), value, jnp.float32)

    def body(i, _):
        ref[pl.ds(i * LANES, LANES)] = v
        return 0

    lax.fori_loop(0, n // LANES, body, 0)


def _sc_count(edges):
    mesh = plsc.VectorSubcoreMesh(**_MESH)

    @functools.partial(
        pl.kernel,
        out_type=jax.ShapeDtypeStruct((NC * 2 * N2,), jnp.float32),
        mesh=mesh,
        scratch_types=[
            [pltpu.VMEM((CHUNK,), jnp.int32) for _ in range(3)],
            [pltpu.VMEM((CHUNK,), jnp.int32) for _ in range(3)],
            pltpu.VMEM((CHUNK,), jnp.float32),
            pltpu.VMEM((SLICE,), jnp.float32),
            pltpu.VMEM_SHARED((N2,), jnp.float32),
            pltpu.VMEM_SHARED((N2,), jnp.float32),
            [pltpu.SemaphoreType.DMA for _ in range(3)],
            [pltpu.SemaphoreType.DMA for _ in range(3)],
        ],
    )
    def k(edges_hbm, out_hbm, srcb, dstb, ones_v, zeros_v, acc_o, acc_i,
          lsem, ssem):
        cid = lax.axis_index("c")
        sid = lax.axis_index("s")
        _fill(ones_v, CHUNK, 1.0)
        _fill(zeros_v, SLICE, 0.0)
        sl = pl.ds(sid * SLICE, SLICE)
        pltpu.sync_copy(zeros_v, acc_o.at[sl])
        pltpu.sync_copy(zeros_v, acc_i.at[sl])
        plsc.subcore_barrier()

        ebase = (cid * NS + sid) * E_PER_W

        def load(i):
            b = i % 3
            base = pl.multiple_of(ebase + i * CHUNK, 8)
            return [
                pltpu.async_copy(edges_hbm.at[pl.ds(base, CHUNK)], srcb[b],
                                 lsem[b]),
                pltpu.async_copy(edges_hbm.at[pl.ds(base + E, CHUNK)], dstb[b],
                                 lsem[b]),
            ]

        def scat(i):
            b = i % 3
            return [
                pltpu.async_copy(ones_v, acc_o.at[srcb[b]], ssem[b], add=True),
                pltpu.async_copy(ones_v, acc_i.at[dstb[b]], ssem[b], add=True),
            ]

        ld = [None] * NCH
        st = [None] * NCH
        ld[0] = load(0)
        ld[1] = load(1)
        for i in range(NCH):
            for c in ld[i]:
                c.wait()
            st[i] = scat(i)
            if i >= 1:
                for c in st[i - 1]:
                    c.wait()
            if i + 2 < NCH:
                ld[i + 2] = load(i + 2)
        for c in st[NCH - 1]:
            c.wait()

        plsc.subcore_barrier()
        obase = pl.multiple_of(cid * 2 * N2 + sid * SLICE, 8)
        pltpu.sync_copy(acc_o.at[sl], out_hbm.at[pl.ds(obase, SLICE)])
        pltpu.sync_copy(acc_i.at[sl], out_hbm.at[pl.ds(obase + N2, SLICE)])

    return k(edges).reshape(NC, 2, N2)


def _tc_feat(x3, cnt):
    def body(x_ref, c_ref, f0_ref, f1_ref, f2_ref):
        deg_o = c_ref[0:1, :] + c_ref[2:3, :]
        so = lax.rsqrt(jnp.maximum(deg_o, 1.0))
        f = x_ref[...] * so
        f0_ref[...] = f[0:1, :]
        f1_ref[...] = f[1:2, :]
        f2_ref[...] = f[2:3, :]

    return pl.pallas_call(
        body,
        out_shape=[jax.ShapeDtypeStruct((1, N2), jnp.float32)] * 3,
    )(x3, cnt)


def _sc_aggregate(edges, f0, f1, f2):
    mesh = plsc.VectorSubcoreMesh(**_MESH)

    @functools.partial(
        pl.kernel,
        out_type=jax.ShapeDtypeStruct((NC * 3 * N2,), jnp.float32),
        mesh=mesh,
        scratch_types=[
            [pltpu.VMEM((CHUNK,), jnp.int32) for _ in range(3)],
            [pltpu.VMEM((CHUNK,), jnp.int32) for _ in range(3)],
            [[pltpu.VMEM((CHUNK,), jnp.float32) for _ in range(3)]
             for _ in range(3)],
            pltpu.VMEM((SLICE,), jnp.float32),
            [pltpu.VMEM_SHARED((N2,), jnp.float32) for _ in range(3)],
            [pltpu.VMEM_SHARED((N2,), jnp.float32) for _ in range(3)],
            [pltpu.SemaphoreType.DMA for _ in range(3)],
            [pltpu.SemaphoreType.DMA for _ in range(3)],
            [pltpu.SemaphoreType.DMA for _ in range(3)],
        ],
    )
    def k(edges_hbm, f0_hbm, f1_hbm, f2_hbm, out_hbm,
          srcb, dstb, vb, zeros_v, acc, fs, lsem, gsem, ssem):
        cid = lax.axis_index("c")
        sid = lax.axis_index("s")
        fh = [f0_hbm, f1_hbm, f2_hbm]
        _fill(zeros_v, SLICE, 0.0)
        sl = pl.ds(sid * SLICE, SLICE)
        for a in acc:
            pltpu.sync_copy(zeros_v, a.at[sl])
        for kk in range(3):
            pltpu.sync_copy(fh[kk].at[sl], fs[kk].at[sl])
        plsc.subcore_barrier()

        ebase = (cid * NS + sid) * E_PER_W

        def load(i):
            b = i % 3
            base = pl.multiple_of(ebase + i * CHUNK, 8)
            return [
                pltpu.async_copy(edges_hbm.at[pl.ds(base, CHUNK)], srcb[b],
                                 lsem[b]),
                pltpu.async_copy(edges_hbm.at[pl.ds(base + E, CHUNK)], dstb[b],
                                 lsem[b]),
            ]

        def gather(i):
            b = i % 3
            return [pltpu.async_copy(fs[kk].at[srcb[b]], vb[kk][b], gsem[b])
                    for kk in range(3)]

        def scat(i):
            b = i % 3
            return [pltpu.async_copy(vb[kk][b], acc[kk].at[dstb[b]], ssem[b],
                                     add=True)
                    for kk in range(3)]

        ld = [None] * NCH
        gt = [None] * NCH
        st = [None] * NCH
        ld[0] = load(0)
        for c in ld[0]:
            c.wait()
        gt[0] = gather(0)
        if NCH > 1:
            ld[1] = load(1)
        for i in range(NCH):
            for c in gt[i]:
                c.wait()
            st[i] = scat(i)
            if i + 1 < NCH:
                for c in ld[i + 1]:
                    c.wait()
                gt[i + 1] = gather(i + 1)
            if i >= 1:
                for c in st[i - 1]:
                    c.wait()
            if i + 2 < NCH:
                ld[i + 2] = load(i + 2)
        for c in st[NCH - 1]:
            c.wait()

        plsc.subcore_barrier()
        obase = pl.multiple_of(cid * 3 * N2 + sid * SLICE, 8)
        for kk in range(3):
            pltpu.sync_copy(acc[kk].at[sl],
                            out_hbm.at[pl.ds(obase + kk * N2, SLICE)])

    return k(edges, f0, f1, f2).reshape(NC, 3, N2)


P2 = 10240
Q = 2048
J = P2 // Q


def _tc_tail(agg4, deg4, w1t, b1, fcwt, fc_b):
    def body(a_ref, d_ref, w_ref, b_ref, fw_ref, fb_ref, o_ref):
        b = pl.program_id(0)
        s = a_ref[0] + a_ref[1]
        si = lax.rsqrt(jnp.maximum(d_ref[0] + d_ref[1], 1.0))
        a = s * si
        h = jnp.dot(w_ref[...], a, preferred_element_type=jnp.float32)
        h = jnp.maximum(h + b_ref[...], 0.0)
        val = jnp.sum(h * fw_ref[...])
        o_ref[b] = jnp.full((8, 128), jax.nn.sigmoid(val + fb_ref[0, 0]),
                            jnp.float32)

    return pl.pallas_call(
        body,
        grid=(B,),
        in_specs=[
            pl.BlockSpec((NC, 3, P2), lambda b: (0, 0, b)),
            pl.BlockSpec((NC, 1, P2), lambda b: (0, 0, b)),
            pl.BlockSpec((F, 3), lambda b: (0, 0)),
            pl.BlockSpec((F, 1), lambda b: (0, 0)),
            pl.BlockSpec((F, P2), lambda b: (0, 0)),
            pl.BlockSpec((1, 1), lambda b: (0, 0)),
        ],
        out_specs=pl.BlockSpec((B, 8, 128), lambda b: (0, 0, 0)),
        out_shape=jax.ShapeDtypeStruct((B, 8, 128), jnp.float32),
    )(agg4, deg4, w1t, b1, fcwt, fc_b)


def kernel(hcal, ecal, trck, edge_index, W1, b1, fc_W, fc_b):
    if edge_index.dtype != jnp.int32:
        edge_index = edge_index.astype(jnp.int32)
    edges = edge_index.reshape(2 * E)

    x3 = jnp.zeros((3, N2), jnp.float32)
    x3 = x3.at[0, :N].set(hcal).at[1, :N].set(ecal).at[2, :N].set(trck)

    cnt = _sc_count(edges)
    f0, f1, f2 = _tc_feat(x3, cnt.reshape(NC * 2, N2))
    aggp = _sc_aggregate(edges, f0.reshape(N2), f1.reshape(N2),
                         f2.reshape(N2))

    agg4 = jnp.pad(aggp[:, :, :N].reshape(NC, 3, B, P),
                   ((0, 0), (0, 0), (0, 0), (0, P2 - P)))
    agg4 = agg4.reshape(NC, 3, B * P2)
    deg4 = jnp.pad(cnt[:, 1:2, :N].reshape(NC, 1, B, P),
                   ((0, 0), (0, 0), (0, 0), (0, P2 - P)))
    deg4 = deg4.reshape(NC, 1, B * P2)
    w1t = W1.T
    b1c = b1.reshape(F, 1)
    fcwt = fc_W.reshape(P, F).T
    fcwt = jnp.pad(fcwt, ((0, 0), (0, P2 - P)))
    fcb = fc_b.reshape(1, 1)
    out = _tc_tail(agg4, deg4, w1t, b1c, fcwt, fcb)
    return out[:, 0, 0:1]
